# baseline (speedup 1.0000x reference)
import functools

import jax
import jax.numpy as jnp
from jax import lax
from jax.experimental import pallas as pl
from jax.experimental.pallas import tpu as pltpu
from jax.experimental.pallas import tpu_sc as plsc

D = 64
NC, NS = 2, 16
NW = NC * NS
CH = 128
NCH = 50
B = NW * NCH * CH

_mesh = plsc.VectorSubcoreMesh(core_axis_name="c", subcore_axis_name="s")


@functools.partial(
    pl.kernel,
    out_type=jax.ShapeDtypeStruct((B, D), jnp.float32),
    mesh=_mesh,
    scratch_types=[
        pltpu.VMEM((NCH, CH), jnp.int32),
        pltpu.VMEM((CH, D), jnp.float32),
        pltpu.SemaphoreType.DMA,
    ],
    compiler_params=pltpu.CompilerParams(use_tc_tiling_on_sc=False),
)
def _lookup(tok_hbm, table_hbm, out_hbm, idx_v, rows_v, sem):
    wid = lax.axis_index("s") * NC + lax.axis_index("c")
    pltpu.sync_copy(tok_hbm.at[wid], idx_v)
    base = wid * (NCH * CH)

    @pl.loop(0, NCH)
    def _chunk(j):
        pltpu.async_copy(table_hbm.at[idx_v.at[j]], rows_v, sem).wait()
        pltpu.sync_copy(rows_v, out_hbm.at[pl.ds(base + j * CH, CH)])


def kernel(tokens, table):
    s0, s1 = tokens.shape
    tok = tokens.reshape(NW, NCH, CH).astype(jnp.int32)
    out = _lookup(tok, table)
    return out.reshape(s0, s1, D)


# speedup vs baseline: 1.0458x; 1.0458x over previous
import functools

import jax
import jax.numpy as jnp
from jax import lax
from jax.experimental import pallas as pl
from jax.experimental.pallas import tpu as pltpu
from jax.experimental.pallas import tpu_sc as plsc

D = 64
NC, NS = 2, 16
NW = NC * NS
CH = 128
NCH = 50
B = NW * NCH * CH
R = 10
AH = 5

_mesh = plsc.VectorSubcoreMesh(core_axis_name="c", subcore_axis_name="s")


@functools.partial(
    pl.kernel,
    out_type=jax.ShapeDtypeStruct((B, D), jnp.float32),
    mesh=_mesh,
    scratch_types=(
        [pltpu.VMEM((NCH, CH), jnp.int32), pltpu.VMEM((R, CH, D), jnp.float32)]
        + [pltpu.SemaphoreType.DMA] * (2 * R)
    ),
    compiler_params=pltpu.CompilerParams(use_tc_tiling_on_sc=False),
)
def _lookup(tok_hbm, table_hbm, out_hbm, idx_v, rows_v, *sems):
    gsem, wsem = sems[:R], sems[R:]
    wid = lax.axis_index("s") * NC + lax.axis_index("c")
    pltpu.sync_copy(tok_hbm.at[wid], idx_v)
    base = wid * (NCH * CH)

    for b in range(AH):
        pltpu.async_copy(table_hbm.at[idx_v.at[b]], rows_v.at[b], gsem[b])

    @pl.loop(0, NCH, step=R)
    def _group(g):
        for b in range(R):
            j = g + b
            pltpu.make_async_copy(
                table_hbm.at[pl.ds(0, CH)], rows_v.at[b], gsem[b]
            ).wait()
            pltpu.async_copy(
                rows_v.at[b], out_hbm.at[pl.ds(base + j * CH, CH)], wsem[b]
            )
            nb = (b + AH) % R
            nxt = j + AH

            @pl.when(nxt < NCH)
            def _fire():
                @pl.when(nxt - R >= 0)
                def _drain_old_write():
                    pltpu.make_async_copy(
                        rows_v.at[nb], out_hbm.at[pl.ds(base, CH)], wsem[nb]
                    ).wait()

                pltpu.async_copy(table_hbm.at[idx_v.at[nxt]], rows_v.at[nb], gsem[nb])

    for b in range(R):
        pltpu.make_async_copy(
            rows_v.at[b], out_hbm.at[pl.ds(base, CH)], wsem[b]
        ).wait()


def kernel(tokens, table):
    s0, s1 = tokens.shape
    tok = tokens.reshape(NW, NCH, CH).astype(jnp.int32)
    out = _lookup(tok, table)
    return out.reshape(s0, s1, D)


# speedup vs baseline: 1.0459x; 1.0001x over previous
import functools

import jax
import jax.numpy as jnp
from jax import lax
from jax.experimental import pallas as pl
from jax.experimental.pallas import tpu as pltpu
from jax.experimental.pallas import tpu_sc as plsc

D = 64
NC, NS = 2, 16
NW = NC * NS
CH = 128
NCH = 50
PW = NCH * CH
B = NW * PW
R = 10
AH = 5

_mesh = plsc.VectorSubcoreMesh(core_axis_name="c", subcore_axis_name="s")


@functools.partial(
    pl.kernel,
    out_type=jax.ShapeDtypeStruct((B, D), jnp.float32),
    mesh=_mesh,
    scratch_types=(
        [pltpu.VMEM((PW,), jnp.int32), pltpu.VMEM((R, CH, D), jnp.float32)]
        + [pltpu.SemaphoreType.DMA] * (2 * R)
    ),
    compiler_params=pltpu.CompilerParams(use_tc_tiling_on_sc=False),
)
def _lookup(tok_hbm, table_hbm, out_hbm, idx_v, rows_v, *sems):
    gsem, wsem = sems[:R], sems[R:]
    wid = lax.axis_index("s") * NC + lax.axis_index("c")
    base = wid * PW
    pltpu.sync_copy(tok_hbm.at[pl.ds(base, PW)], idx_v)

    for b in range(AH):
        pltpu.async_copy(
            table_hbm.at[idx_v.at[pl.ds(b * CH, CH)]], rows_v.at[b], gsem[b]
        )

    @pl.loop(0, NCH, step=R)
    def _group(g):
        for b in range(R):
            j = g + b
            pltpu.make_async_copy(
                table_hbm.at[pl.ds(0, CH)], rows_v.at[b], gsem[b]
            ).wait()
            pltpu.async_copy(
                rows_v.at[b], out_hbm.at[pl.ds(base + j * CH, CH)], wsem[b]
            )
            nb = (b + AH) % R
            nxt = j + AH

            @pl.when(nxt < NCH)
            def _fire():
                @pl.when(nxt - R >= 0)
                def _drain_old_write():
                    pltpu.make_async_copy(
                        rows_v.at[nb], out_hbm.at[pl.ds(base, CH)], wsem[nb]
                    ).wait()

                pltpu.async_copy(
                    table_hbm.at[idx_v.at[pl.ds(nxt * CH, CH)]],
                    rows_v.at[nb],
                    gsem[nb],
                )

    for b in range(R):
        pltpu.make_async_copy(
            rows_v.at[b], out_hbm.at[pl.ds(base, CH)], wsem[b]
        ).wait()


def kernel(tokens, table):
    s0, s1 = tokens.shape
    tok = tokens.reshape(B).astype(jnp.int32)
    out = _lookup(tok, table)
    return out.reshape(s0, s1, D)
